# Initial kernel scaffold; baseline (speedup 1.0000x reference)
import jax
import jax.numpy as jnp
from jax.experimental import pallas as pl


def kernel(t, genres, t_table, style_table):
    raise NotImplementedError("write your pallas kernel here")


# speedup vs baseline: 8.7250x; 8.7250x over previous
import functools

import jax
import jax.numpy as jnp
from jax.experimental import pallas as pl

B = 16384
G = 50
G_PAD = 56
T_ROWS = 1024
S_ROWS = 128
D = 64
R = 128


def _body(t_ref, gf_ref, ttab_ref, stab_ref, out_ref):
    tb = t_ref[...]
    oh_t = (tb == jax.lax.broadcasted_iota(jnp.int32, (R, T_ROWS), 1)).astype(jnp.bfloat16)
    cond = jnp.dot(oh_t, ttab_ref[...], preferred_element_type=jnp.float32)

    gf = gf_ref[...]
    s = R * G_PAD
    oh_s = (gf == jax.lax.broadcasted_iota(jnp.int32, (s, S_ROWS), 1)).astype(jnp.bfloat16)
    rows = jnp.dot(oh_s, stab_ref[...], preferred_element_type=jnp.float32)
    styles = jnp.sum(rows.reshape(R, G_PAD, D), axis=1)

    out_ref[...] = jnp.concatenate([cond, styles], axis=1)


@jax.jit
def kernel(t, genres, t_table, style_table):
    t2 = t.reshape(B, 1).astype(jnp.int32)
    gpad = jnp.full((B, G_PAD - G), S_ROWS - 1, dtype=genres.dtype)
    gf = jnp.concatenate([genres, gpad], axis=1).reshape(B * G_PAD, 1).astype(jnp.int32)
    ttab = jnp.zeros((T_ROWS, D), jnp.bfloat16).at[: t_table.shape[0]].set(
        t_table.astype(jnp.bfloat16))
    stab = jnp.zeros((S_ROWS, D), jnp.bfloat16).at[: style_table.shape[0]].set(
        style_table.astype(jnp.bfloat16))

    nb = B // R
    out = pl.pallas_call(
        _body,
        grid=(nb,),
        in_specs=[
            pl.BlockSpec((R, 1), lambda i: (i, 0)),
            pl.BlockSpec((R * G_PAD, 1), lambda i: (i, 0)),
            pl.BlockSpec((T_ROWS, D), lambda i: (0, 0)),
            pl.BlockSpec((S_ROWS, D), lambda i: (0, 0)),
        ],
        out_specs=pl.BlockSpec((R, 128), lambda i: (i, 0)),
        out_shape=jax.ShapeDtypeStruct((B, 128), jnp.float32),
    )(t2, gf, ttab, stab)
    return out[:, :, None]


# speedup vs baseline: 42.8520x; 4.9114x over previous
import functools

import jax
import jax.numpy as jnp
from jax import lax
from jax.experimental import pallas as pl
from jax.experimental.pallas import tpu as pltpu
from jax.experimental.pallas import tpu_sc as plsc

B = 16384
G = 50
D = 64
C_PAD = 112
NW = 32
RB = B // NW
N_GRP = RB // 16


def _sc_body(t_hbm, genres_hbm, ttab_hbm, trows_hbm, counts_hbm,
             idx_v, gen_v, trows_v, counts_v, sem):
    wid = lax.axis_index("s") * 2 + lax.axis_index("c")
    base = wid * RB

    pltpu.sync_copy(t_hbm.at[pl.ds(wid * 4, 4)], idx_v)
    copies = [
        pltpu.async_copy(ttab_hbm.at[idx_v.at[j]],
                         trows_v.at[pl.ds(j * 128, 128)], sem)
        for j in range(4)
    ]
    pltpu.sync_copy(genres_hbm.at[pl.ds(base * G, RB * G)], gen_v)

    iota16 = lax.broadcasted_iota(jnp.int32, (16,), 0)
    ones16 = jnp.ones((16,), jnp.float32)
    zeros16 = jnp.zeros((16,), jnp.float32)

    def grp(i, carry):
        rows16 = i * 16 + iota16
        gbase = rows16 * G
        cbase = rows16 * C_PAD
        for r in range(16):
            row = (i * 16 + r) * C_PAD
            for c in range(C_PAD // 16):
                counts_v[pl.ds(row + c * 16, 16)] = zeros16
        for s in range(G):
            g16 = plsc.load_gather(gen_v, [gbase + s])
            plsc.addupdate_scatter(counts_v, [cbase + g16], ones16)
        return carry

    lax.fori_loop(0, N_GRP, grp, 0)

    pltpu.sync_copy(counts_v, counts_hbm.at[pl.ds(base * C_PAD, RB * C_PAD)])
    for cp in copies:
        cp.wait()
    pltpu.sync_copy(trows_v, trows_hbm.at[pl.ds(base, RB)])


def _tc_body(trows_ref, counts_ref, stab_ref, out_ref):
    styles = jnp.dot(counts_ref[...].astype(jnp.bfloat16), stab_ref[...],
                     preferred_element_type=jnp.float32)
    out_ref[...] = jnp.concatenate([trows_ref[...], styles], axis=1)


@jax.jit
def kernel(t, genres, t_table, style_table):
    t2 = t.reshape(B // 128, 128).astype(jnp.int32)
    genres_flat = genres.astype(jnp.int32).reshape(B * G)
    stab = jnp.zeros((C_PAD, D), jnp.bfloat16).at[: style_table.shape[0]].set(
        style_table.astype(jnp.bfloat16))

    mesh = plsc.VectorSubcoreMesh(core_axis_name="c", subcore_axis_name="s")
    trows, counts = pl.kernel(
        _sc_body,
        mesh=mesh,
        compiler_params=pltpu.CompilerParams(
            needs_layout_passes=False, use_tc_tiling_on_sc=False),
        out_type=[
            jax.ShapeDtypeStruct((B, D), jnp.float32),
            jax.ShapeDtypeStruct((B * C_PAD,), jnp.float32),
        ],
        scratch_types=[
            pltpu.VMEM((4, 128), jnp.int32),
            pltpu.VMEM((RB * G,), jnp.int32),
            pltpu.VMEM((RB, D), jnp.float32),
            pltpu.VMEM((RB * C_PAD,), jnp.float32),
            pltpu.SemaphoreType.DMA,
        ],
    )(t2, genres_flat, t_table)
    counts = counts.reshape(B, C_PAD)

    rt = 512
    out = pl.pallas_call(
        _tc_body,
        grid=(B // rt,),
        in_specs=[
            pl.BlockSpec((rt, D), lambda i: (i, 0)),
            pl.BlockSpec((rt, C_PAD), lambda i: (i, 0)),
            pl.BlockSpec((C_PAD, D), lambda i: (0, 0)),
        ],
        out_specs=pl.BlockSpec((rt, 128), lambda i: (i, 0)),
        out_shape=jax.ShapeDtypeStruct((B, 128), jnp.float32),
    )(trows, counts, stab)
    return out[:, :, None]


# speedup vs baseline: 52.2610x; 1.2196x over previous
import functools

import jax
import jax.numpy as jnp
from jax import lax
from jax.experimental import pallas as pl
from jax.experimental.pallas import tpu as pltpu
from jax.experimental.pallas import tpu_sc as plsc

B = 16384
G = 50
D = 64
C_PAD = 128
NW = 32
RB = B // NW
N_GRP = RB // 16


def _sc_body(t_hbm, genres_hbm, ttab_hbm, trows_hbm, counts_hbm,
             idx_v, gen_v, trows_v, counts_v, sem):
    wid = lax.axis_index("s") * 2 + lax.axis_index("c")
    base = wid * RB

    pltpu.sync_copy(t_hbm.at[pl.ds(wid * 4, 4)], idx_v)
    copies = [
        pltpu.async_copy(ttab_hbm.at[idx_v.at[j]],
                         trows_v.at[pl.ds(j * 128, 128)], sem)
        for j in range(4)
    ]
    pltpu.sync_copy(genres_hbm.at[pl.ds(base, RB)], gen_v)

    iota16 = lax.broadcasted_iota(jnp.int32, (16,), 0)
    ones16 = jnp.ones((16,), jnp.float32)
    zeros16 = jnp.zeros((16,), jnp.float32)

    def grp(i, carry):
        rows16 = i * 16 + iota16
        for r in range(16):
            row = i * 16 + r
            for c in range(C_PAD // 16):
                counts_v[row, pl.ds(c * 16, 16)] = zeros16
        for s in range(G):
            g16 = plsc.load_gather(gen_v, [rows16, jnp.full((16,), s, jnp.int32)])
            plsc.addupdate_scatter(counts_v, [rows16, g16], ones16)
        return carry

    lax.fori_loop(0, N_GRP, grp, 0)

    pltpu.sync_copy(counts_v, counts_hbm.at[pl.ds(base, RB)])
    for cp in copies:
        cp.wait()
    pltpu.sync_copy(trows_v, trows_hbm.at[pl.ds(base, RB), pl.ds(0, D)])


def _tc_body(trows_ref, counts_ref, stab_ref, out_ref):
    styles = jnp.dot(counts_ref[...].astype(jnp.bfloat16), stab_ref[...],
                     preferred_element_type=jnp.float32)
    out_ref[...] = jnp.concatenate([trows_ref[:, :D], styles], axis=1)


@jax.jit
def kernel(t, genres, t_table, style_table):
    t2 = t.reshape(B // 128, 128).astype(jnp.int32)
    genres = genres.astype(jnp.int32)
    stab = jnp.zeros((C_PAD, D), jnp.bfloat16).at[: style_table.shape[0]].set(
        style_table.astype(jnp.bfloat16))

    mesh = plsc.VectorSubcoreMesh(core_axis_name="c", subcore_axis_name="s")
    trows, counts = pl.kernel(
        _sc_body,
        mesh=mesh,
        compiler_params=pltpu.CompilerParams(
            needs_layout_passes=False, use_tc_tiling_on_sc=False),
        out_type=[
            jax.ShapeDtypeStruct((B, 128), jnp.float32),
            jax.ShapeDtypeStruct((B, C_PAD), jnp.float32),
        ],
        scratch_types=[
            pltpu.VMEM((4, 128), jnp.int32),
            pltpu.VMEM((RB, G), jnp.int32),
            pltpu.VMEM((RB, D), jnp.float32),
            pltpu.VMEM((RB, C_PAD), jnp.float32),
            pltpu.SemaphoreType.DMA,
        ],
    )(t2, genres, t_table)

    rt = 2048
    out = pl.pallas_call(
        _tc_body,
        grid=(B // rt,),
        in_specs=[
            pl.BlockSpec((rt, 128), lambda i: (i, 0)),
            pl.BlockSpec((rt, C_PAD), lambda i: (i, 0)),
            pl.BlockSpec((C_PAD, D), lambda i: (0, 0)),
        ],
        out_specs=pl.BlockSpec((rt, 128), lambda i: (i, 0)),
        out_shape=jax.ShapeDtypeStruct((B, 128), jnp.float32),
    )(trows, counts, stab)
    return out[:, :, None]


# speedup vs baseline: 78.4609x; 1.5013x over previous
import functools

import jax
import jax.numpy as jnp
from jax import lax
from jax.experimental import pallas as pl
from jax.experimental.pallas import tpu as pltpu
from jax.experimental.pallas import tpu_sc as plsc

B = 16384
G = 50
D = 64
C_PAD = 128
NW = 32
RB = B // NW
N_GRP = RB // 16
N_CHUNK = 4


def _sc_body(t_hbm, genres_hbm, ttab_hbm, trows_hbm, counts_hbm,
             idx_v, gen_v, trows_v, counts_v, sem, gsem, csem):
    wid = lax.axis_index("s") * 2 + lax.axis_index("c")
    base = wid * RB

    gen_cp = pltpu.async_copy(genres_hbm.at[:, pl.ds(base, RB)], gen_v, gsem)
    pltpu.sync_copy(t_hbm.at[pl.ds(wid * 4, 4)], idx_v)
    copies = [
        pltpu.async_copy(ttab_hbm.at[idx_v.at[j]],
                         trows_v.at[pl.ds(j * 128, 128)], sem)
        for j in range(4)
    ]

    iota16 = lax.broadcasted_iota(jnp.int32, (16,), 0)
    ones16 = jnp.ones((16,), jnp.float32)
    zeros16 = jnp.zeros((16,), jnp.float32)

    def zrow(r, carry):
        for c in range(C_PAD // 16):
            counts_v[r, pl.ds(c * 16, 16)] = zeros16
        return carry

    lax.fori_loop(0, RB, zrow, 0)
    gen_cp.wait()

    grp_per_chunk = N_GRP // N_CHUNK
    rows_per_chunk = RB // N_CHUNK

    def grp(i, carry):
        rows16 = i * 16 + iota16
        for s in range(G):
            g16 = gen_v[s, pl.ds(i * 16, 16)]
            plsc.addupdate_scatter(counts_v, [rows16, g16], ones16)
        return carry

    ccopies = []
    for ch in range(N_CHUNK):
        lax.fori_loop(ch * grp_per_chunk, (ch + 1) * grp_per_chunk, grp, 0)
        r0 = ch * rows_per_chunk
        ccopies.append(pltpu.async_copy(
            counts_v.at[pl.ds(r0, rows_per_chunk)],
            counts_hbm.at[pl.ds(base + r0, rows_per_chunk)], csem))

    for cp in copies:
        cp.wait()
    pltpu.sync_copy(trows_v, trows_hbm.at[pl.ds(base, RB), pl.ds(0, D)])
    for cp in ccopies:
        cp.wait()


def _tc_body(trows_ref, counts_ref, stab_ref, out_ref):
    styles = jnp.dot(counts_ref[...].astype(jnp.bfloat16), stab_ref[...],
                     preferred_element_type=jnp.float32)
    out_ref[...] = jnp.concatenate([trows_ref[:, :D], styles], axis=1)


@jax.jit
def kernel(t, genres, t_table, style_table):
    t2 = t.reshape(B // 128, 128).astype(jnp.int32)
    genres_t = genres.astype(jnp.int32).T
    stab = jnp.zeros((C_PAD, D), jnp.bfloat16).at[: style_table.shape[0]].set(
        style_table.astype(jnp.bfloat16))

    mesh = plsc.VectorSubcoreMesh(core_axis_name="c", subcore_axis_name="s")
    trows, counts = pl.kernel(
        _sc_body,
        mesh=mesh,
        compiler_params=pltpu.CompilerParams(
            needs_layout_passes=False, use_tc_tiling_on_sc=False),
        out_type=[
            jax.ShapeDtypeStruct((B, 128), jnp.float32),
            jax.ShapeDtypeStruct((B, C_PAD), jnp.float32),
        ],
        scratch_types=[
            pltpu.VMEM((4, 128), jnp.int32),
            pltpu.VMEM((G, RB), jnp.int32),
            pltpu.VMEM((RB, D), jnp.float32),
            pltpu.VMEM((RB, C_PAD), jnp.float32),
            pltpu.SemaphoreType.DMA,
            pltpu.SemaphoreType.DMA,
            pltpu.SemaphoreType.DMA,
        ],
    )(t2, genres_t, t_table)

    rt = 4096
    out = pl.pallas_call(
        _tc_body,
        grid=(B // rt,),
        in_specs=[
            pl.BlockSpec((rt, 128), lambda i: (i, 0)),
            pl.BlockSpec((rt, C_PAD), lambda i: (i, 0)),
            pl.BlockSpec((C_PAD, D), lambda i: (0, 0)),
        ],
        out_specs=pl.BlockSpec((rt, 128), lambda i: (i, 0)),
        out_shape=jax.ShapeDtypeStruct((B, 128), jnp.float32),
    )(trows, counts, stab)
    return out[:, :, None]
